# baseline (speedup 1.0000x reference)
import functools
import jax
import jax.numpy as jnp
from jax import lax
from jax.experimental import pallas as pl
from jax.experimental.pallas import tpu as pltpu
from jax.experimental.pallas import tpu_sc as plsc

_IMG_DIM = 608.0
_NG = 76
_HW = _NG * _NG
_A = 3
_TASK = 5 * _HW
_STRIDE = _IMG_DIM / _NG
_AW = (10.0, 16.0, 33.0)
_AH = (13.0, 30.0, 23.0)
_CROWS = 384
_NPAIR = 8
_LAST_R0 = _HW - _CROWS


def _sc_body(x_hbm, out_hbm, in0, cb0, cb1, si0, sc0, sc1):
    n_tasks = 6
    wid = lax.axis_index("s") * 2 + lax.axis_index("c")
    iota = lax.iota(jnp.int32, 16)
    ins = (in0, in0)
    isems = (si0, si0)
    cbufs = (cb0, cb1)
    csems = (sc0, sc1)
    base_task = wid * n_tasks

    def in_copy(t):
        task = base_task + t
        return pltpu.async_copy(
            x_hbm.at[pl.ds(task * _TASK, _TASK)],
            ins[t % 2], isems[t % 2])

    def drain(par):
        pltpu.make_async_copy(
            cbufs[par], out_hbm.at[pl.ds(0, _CROWS), :], csems[par]).wait()

    for t in range(n_tasks):
        in_copy(t).wait()
        ib = ins[t % 2]
        task = base_task + t
        a = task % _A
        aw = jnp.where(a == 0, _AW[0],
                       jnp.where(a == 1, _AW[1], _AW[2])).astype(jnp.float32)
        ah = jnp.where(a == 0, _AH[0],
                       jnp.where(a == 1, _AH[1], _AH[2])).astype(jnp.float32)

        def pair_body(j, carry):
            for par in range(2):
                c = 2 * j + par
                r0 = jnp.minimum(c * _CROWS, _LAST_R0)

                def wait_prev():
                    drain(par)

                if t == 0:
                    pl.when(j > 0)(wait_prev)
                else:
                    wait_prev()
                cb = cbufs[par]

                @plsc.parallel_loop(0, _CROWS // 16, unroll=4)
                def _chunk(i):
                    s = r0 + i * 16
                    p = s + iota
                    gx = lax.rem(p, _NG).astype(jnp.float32)
                    gy = lax.div(p, _NG).astype(jnp.float32)
                    tx = ib[pl.ds(s, 16)]
                    ty = ib[pl.ds(_HW + s, 16)]
                    tw = ib[pl.ds(2 * _HW + s, 16)]
                    th = ib[pl.ds(3 * _HW + s, 16)]
                    tc = ib[pl.ds(4 * _HW + s, 16)]
                    one = jnp.float32(1.0)
                    bx = (one / (one + jnp.exp(-tx)) + gx) * _STRIDE
                    by = (one / (one + jnp.exp(-ty)) + gy) * _STRIDE
                    bw = jnp.exp(tw) * aw
                    bh = jnp.exp(th) * ah
                    cf = one / (one + jnp.exp(-tc))
                    row = i * 16 + iota
                    zero = iota - iota
                    for attr, val in ((0, bx), (1, by), (2, bw), (3, bh),
                                      (4, cf)):
                        plsc.store_scatter(cb, [row, zero + attr], val)

                pltpu.make_async_copy(
                    cb, out_hbm.at[pl.ds(task * _HW + r0, _CROWS), :],
                    csems[par]).start()
            return carry

        lax.fori_loop(0, _NPAIR, pair_body, 0)
    drain(0)
    drain(1)


def kernel(x):
    B, C, nG, _ = x.shape
    xf = x.reshape(-1)
    run = functools.partial(
        pl.kernel,
        out_type=jax.ShapeDtypeStruct((B * _A * _HW, 5), jnp.float32),
        mesh=plsc.VectorSubcoreMesh(core_axis_name="c", subcore_axis_name="s"),
        scratch_types=[
            pltpu.VMEM((_TASK,), jnp.float32),
            pltpu.VMEM((_CROWS, 5), jnp.float32),
            pltpu.VMEM((_CROWS, 5), jnp.float32),
            pltpu.SemaphoreType.DMA,
            pltpu.SemaphoreType.DMA,
            pltpu.SemaphoreType.DMA,
        ],
        compiler_params=pltpu.CompilerParams(needs_layout_passes=False),
    )(_sc_body)
    out = run(xf)
    return out.reshape(B, _A * _HW, 5)


# speedup vs baseline: 1.0124x; 1.0124x over previous
import functools
import jax
import jax.numpy as jnp
from jax import lax
from jax.experimental import pallas as pl
from jax.experimental.pallas import tpu as pltpu
from jax.experimental.pallas import tpu_sc as plsc

_IMG_DIM = 608.0
_NG = 76
_HW = _NG * _NG
_A = 3
_TASK = 5 * _HW
_STRIDE = _IMG_DIM / _NG
_AW = (10.0, 16.0, 33.0)
_AH = (13.0, 30.0, 23.0)
_CROWS = 256
_NPAIR = 12
_LAST_R0 = _HW - _CROWS


def _sc_body(x_hbm, out_hbm, in0, in1, cb0, cb1, si0, si1, sc0, sc1):
    n_tasks = 6
    wid = lax.axis_index("s") * 2 + lax.axis_index("c")
    iota = lax.iota(jnp.int32, 16)
    ins = (in0, in1)
    isems = (si0, si1)
    cbufs = (cb0, cb1)
    csems = (sc0, sc1)
    base_task = wid * n_tasks

    def in_copy(t):
        task = base_task + t
        return pltpu.async_copy(
            x_hbm.at[pl.ds(task * _TASK, _TASK)],
            ins[t % 2], isems[t % 2])

    def drain(par):
        pltpu.make_async_copy(
            cbufs[par], out_hbm.at[pl.ds(0, _CROWS), :], csems[par]).wait()

    in_dmas = {0: in_copy(0)}
    for t in range(n_tasks):
        if t + 1 < n_tasks:
            in_dmas[t + 1] = in_copy(t + 1)
        in_dmas[t].wait()
        ib = ins[t % 2]
        task = base_task + t
        a = task % _A
        aw = jnp.where(a == 0, _AW[0],
                       jnp.where(a == 1, _AW[1], _AW[2])).astype(jnp.float32)
        ah = jnp.where(a == 0, _AH[0],
                       jnp.where(a == 1, _AH[1], _AH[2])).astype(jnp.float32)

        def pair_body(j, carry):
            for par in range(2):
                c = 2 * j + par
                r0 = jnp.minimum(c * _CROWS, _LAST_R0)

                def wait_prev():
                    drain(par)

                if t == 0:
                    pl.when(j > 0)(wait_prev)
                else:
                    wait_prev()
                cb = cbufs[par]

                @plsc.parallel_loop(0, _CROWS // 16, unroll=4)
                def _chunk(i):
                    s = r0 + i * 16
                    p = s + iota
                    gx = lax.rem(p, _NG).astype(jnp.float32)
                    gy = lax.div(p, _NG).astype(jnp.float32)
                    tx = ib[pl.ds(s, 16)]
                    ty = ib[pl.ds(_HW + s, 16)]
                    tw = ib[pl.ds(2 * _HW + s, 16)]
                    th = ib[pl.ds(3 * _HW + s, 16)]
                    tc = ib[pl.ds(4 * _HW + s, 16)]
                    one = jnp.float32(1.0)
                    bx = (one / (one + jnp.exp(-tx)) + gx) * _STRIDE
                    by = (one / (one + jnp.exp(-ty)) + gy) * _STRIDE
                    bw = jnp.exp(tw) * aw
                    bh = jnp.exp(th) * ah
                    cf = one / (one + jnp.exp(-tc))
                    row = i * 16 + iota
                    zero = iota - iota
                    for attr, val in ((0, bx), (1, by), (2, bw), (3, bh),
                                      (4, cf)):
                        plsc.store_scatter(cb, [row, zero + attr], val)

                pltpu.make_async_copy(
                    cb, out_hbm.at[pl.ds(task * _HW + r0, _CROWS), :],
                    csems[par]).start()
            return carry

        lax.fori_loop(0, _NPAIR, pair_body, 0)
    drain(0)
    drain(1)


def kernel(x):
    B, C, nG, _ = x.shape
    xf = x.reshape(-1)
    run = functools.partial(
        pl.kernel,
        out_type=jax.ShapeDtypeStruct((B * _A * _HW, 5), jnp.float32),
        mesh=plsc.VectorSubcoreMesh(core_axis_name="c", subcore_axis_name="s"),
        scratch_types=[
            pltpu.VMEM((_TASK,), jnp.float32),
            pltpu.VMEM((_TASK,), jnp.float32),
            pltpu.VMEM((_CROWS, 5), jnp.float32),
            pltpu.VMEM((_CROWS, 5), jnp.float32),
            pltpu.SemaphoreType.DMA,
            pltpu.SemaphoreType.DMA,
            pltpu.SemaphoreType.DMA,
            pltpu.SemaphoreType.DMA,
        ],
        compiler_params=pltpu.CompilerParams(needs_layout_passes=False),
    )(_sc_body)
    out = run(xf)
    return out.reshape(B, _A * _HW, 5)
